# Initial kernel scaffold; baseline (speedup 1.0000x reference)
import jax
import jax.numpy as jnp
from jax.experimental import pallas as pl


def kernel(x_author, x_paper, ei_writes, ei_rev, Wk, bk, Wq, bq, Wv, bv, Wa, ba, skip, Arel, Mrel, prel):
    raise NotImplementedError("write your pallas kernel here")


# speedup vs baseline: 16.4772x; 16.4772x over previous
import functools

import jax
import jax.numpy as jnp
import numpy as np
from jax import lax
from jax.experimental import pallas as pl
from jax.experimental.pallas import tpu as pltpu
from jax.experimental.pallas import tpu_sc as plsc

H = 8
D = 16
C = 128
N = 10000
E = 160000

NC = 2
NS = 16
NW = NC * NS

NPAD = 10240
EPW = 5120
E_PAD = EPW * NW
CH = 128
NCHUNK = EPW // CH
CH_S = 64
NCHUNK_S = EPW // CH_S
ROWS_PER_SUB = NPAD // NS
_BLK = 512


def _proj_body(x_ref, wq_ref, bq_ref, wk_ref, bk_ref, wv_ref, bv_ref,
               bda_ref, bdm_ref, q_ref, k_ref, v_ref):
    x = x_ref[...]
    f32 = jnp.float32
    q_ref[...] = jnp.dot(x, wq_ref[...], preferred_element_type=f32) + bq_ref[...]
    kp = jnp.dot(x, wk_ref[...], preferred_element_type=f32) + bk_ref[...]
    vp = jnp.dot(x, wv_ref[...], preferred_element_type=f32) + bv_ref[...]
    k_ref[...] = jnp.dot(kp, bda_ref[...], preferred_element_type=f32)
    v_ref[...] = jnp.dot(vp, bdm_ref[...], preferred_element_type=f32)


def _proj(x, wq, bq, wk, bk, wv, bv, bda, bdm):
    row = pl.BlockSpec((_BLK, C), lambda i: (i, 0))
    full = pl.BlockSpec((C, C), lambda i: (0, 0))
    bias = pl.BlockSpec((1, C), lambda i: (0, 0))
    out = jax.ShapeDtypeStruct((NPAD, C), jnp.float32)
    return pl.pallas_call(
        _proj_body,
        grid=(NPAD // _BLK,),
        in_specs=[row, full, bias, full, bias, full, bias, full, full],
        out_specs=[row, row, row],
        out_shape=[out, out, out],
    )(x, wq, bq, wk, bk, wv, bv, bda, bdm)


def _gather_body(q_hbm, k_hbm, v_hbm, src_hbm, dst_hbm,
                 qg_out, kg_out, vg_out,
                 sidx, didx, qbuf, kbuf, vbuf, sem1, sem2, sem3):
    cid = lax.axis_index("c")
    sid = lax.axis_index("s")
    wid = cid * NS + sid
    base = wid * EPW

    def chunk_body(ci, carry):
        off = base + ci * CH
        pltpu.sync_copy(src_hbm.at[pl.ds(off, CH)], sidx)
        pltpu.sync_copy(dst_hbm.at[pl.ds(off, CH)], didx)
        cp1 = pltpu.async_copy(k_hbm.at[sidx], kbuf, sem1)
        cp2 = pltpu.async_copy(q_hbm.at[didx], qbuf, sem2)
        cp3 = pltpu.async_copy(v_hbm.at[sidx], vbuf, sem3)
        cp1.wait()
        cp2.wait()
        cp3.wait()
        pltpu.sync_copy(qbuf, qg_out.at[pl.ds(off, CH)])
        pltpu.sync_copy(kbuf, kg_out.at[pl.ds(off, CH)])
        pltpu.sync_copy(vbuf, vg_out.at[pl.ds(off, CH)])
        return carry

    lax.fori_loop(0, NCHUNK, chunk_body, 0)


_ROWS = jax.ShapeDtypeStruct((E_PAD, C), jnp.float32)
_gather_kernel = functools.partial(
    pl.kernel,
    out_type=[_ROWS, _ROWS, _ROWS],
    mesh=plsc.VectorSubcoreMesh(core_axis_name="c", subcore_axis_name="s"),
    scratch_types=[
        pltpu.VMEM((CH,), jnp.int32),
        pltpu.VMEM((CH,), jnp.int32),
        pltpu.VMEM((CH, C), jnp.float32),
        pltpu.VMEM((CH, C), jnp.float32),
        pltpu.VMEM((CH, C), jnp.float32),
        pltpu.SemaphoreType.DMA,
        pltpu.SemaphoreType.DMA,
        pltpu.SemaphoreType.DMA,
    ],
)(_gather_body)


def _exwv_body(qg_ref, kg_ref, vg_ref, r16t_ref, r16_ref, wv_ref, ex_ref):
    f32 = jnp.float32
    i = pl.program_id(0)
    prod = qg_ref[...] * kg_ref[...]
    s16 = jnp.dot(prod, r16t_ref[...], preferred_element_type=f32)
    rows = i * _BLK + lax.broadcasted_iota(jnp.int32, (_BLK, 16), 0)
    cols = lax.broadcasted_iota(jnp.int32, (_BLK, 16), 1)
    ex = jnp.where((rows < E) & (cols < H), jnp.exp(s16), 0.0)
    ex_ref[...] = ex
    wv_ref[...] = vg_ref[...] * jnp.dot(ex, r16_ref[...],
                                        preferred_element_type=f32)


def _exwv(qg, kg, vg, r16t, r16):
    row = pl.BlockSpec((_BLK, C), lambda i: (i, 0))
    return pl.pallas_call(
        _exwv_body,
        grid=(E_PAD // _BLK,),
        in_specs=[row, row, row,
                  pl.BlockSpec((C, 16), lambda i: (0, 0)),
                  pl.BlockSpec((16, C), lambda i: (0, 0))],
        out_specs=[row, pl.BlockSpec((_BLK, 16), lambda i: (i, 0))],
        out_shape=[jax.ShapeDtypeStruct((E_PAD, C), jnp.float32),
                   jax.ShapeDtypeStruct((E_PAD, 16), jnp.float32)],
    )(qg, kg, vg, r16t, r16)


def _scatter_body(wv_hbm, ex_hbm, dst_hbm,
                  agg_out, den_out,
                  didx, wvbuf, exbuf, agg_sh, den_sh, sem1, sem2):
    cid = lax.axis_index("c")
    sid = lax.axis_index("s")
    wid = cid * NS + sid
    base = wid * EPW
    r0 = sid * ROWS_PER_SUB
    zv = jnp.zeros((16,), jnp.float32)

    def zrow(i, carry):
        for j in range(C // 16):
            wvbuf[i, pl.ds(j * 16, 16)] = zv
        exbuf[i] = zv
        return carry

    lax.fori_loop(0, CH_S, zrow, 0)

    def zstripe(j, carry):
        pltpu.sync_copy(wvbuf, agg_sh.at[pl.ds(r0 + j * CH_S, CH_S)])
        pltpu.sync_copy(exbuf, den_sh.at[pl.ds(r0 + j * CH_S, CH_S)])
        return carry

    lax.fori_loop(0, ROWS_PER_SUB // CH_S, zstripe, 0)
    plsc.subcore_barrier()

    def chunk_body(ci, carry):
        off = base + ci * CH_S
        pltpu.sync_copy(dst_hbm.at[pl.ds(off, CH_S)], didx)
        cp1 = pltpu.async_copy(wv_hbm.at[pl.ds(off, CH_S)], wvbuf, sem1)
        cp2 = pltpu.async_copy(ex_hbm.at[pl.ds(off, CH_S)], exbuf, sem2)
        cp1.wait()
        cp2.wait()
        pltpu.sync_copy(wvbuf, agg_sh.at[didx], add=True)
        pltpu.sync_copy(exbuf, den_sh.at[didx], add=True)
        return carry

    lax.fori_loop(0, NCHUNK_S, chunk_body, 0)
    plsc.subcore_barrier()

    def flush(j, carry):
        rr = r0 + j * CH_S
        pltpu.sync_copy(agg_sh.at[pl.ds(rr, CH_S)], wvbuf)
        pltpu.sync_copy(wvbuf, agg_out.at[cid, pl.ds(rr, CH_S)])
        pltpu.sync_copy(den_sh.at[pl.ds(rr, CH_S)], exbuf)
        pltpu.sync_copy(exbuf, den_out.at[cid, pl.ds(rr, CH_S)])
        return carry

    lax.fori_loop(0, ROWS_PER_SUB // CH_S, flush, 0)


_scatter_kernel = functools.partial(
    pl.kernel,
    out_type=[jax.ShapeDtypeStruct((NC, NPAD, C), jnp.float32),
              jax.ShapeDtypeStruct((NC, NPAD, 16), jnp.float32)],
    mesh=plsc.VectorSubcoreMesh(core_axis_name="c", subcore_axis_name="s"),
    scratch_types=[
        pltpu.VMEM((CH_S,), jnp.int32),
        pltpu.VMEM((CH_S, C), jnp.float32),
        pltpu.VMEM((CH_S, 16), jnp.float32),
        pltpu.VMEM_SHARED((NPAD, C), jnp.float32),
        pltpu.VMEM_SHARED((NPAD, 16), jnp.float32),
        pltpu.SemaphoreType.DMA,
        pltpu.SemaphoreType.DMA,
    ],
)(_scatter_body)


def _out_body(aggp_ref, denp_ref, x_ref, r16_ref, wa_ref, ba_ref, skip_ref,
              o_ref):
    f32 = jnp.float32
    agg = aggp_ref[0] + aggp_ref[1]
    den = denp_ref[0] + denp_ref[1]
    den_rep = jnp.dot(den, r16_ref[...], preferred_element_type=f32)
    norm = jnp.where(den_rep > 0, agg / den_rep, 0.0)
    g = jax.nn.gelu(norm)
    o = jnp.dot(g, wa_ref[...], preferred_element_type=f32) + ba_ref[...]
    beta = jax.nn.sigmoid(skip_ref[...])
    o_ref[...] = beta * o + (1.0 - beta) * x_ref[...]


def _out_stage(aggp, denp, x, r16, wa, ba, skip11):
    return pl.pallas_call(
        _out_body,
        grid=(NPAD // _BLK,),
        in_specs=[
            pl.BlockSpec((NC, _BLK, C), lambda i: (0, i, 0)),
            pl.BlockSpec((NC, _BLK, 16), lambda i: (0, i, 0)),
            pl.BlockSpec((_BLK, C), lambda i: (i, 0)),
            pl.BlockSpec((16, C), lambda i: (0, 0)),
            pl.BlockSpec((C, C), lambda i: (0, 0)),
            pl.BlockSpec((1, C), lambda i: (0, 0)),
            pl.BlockSpec((1, 1), lambda i: (0, 0)),
        ],
        out_specs=pl.BlockSpec((_BLK, C), lambda i: (i, 0)),
        out_shape=jax.ShapeDtypeStruct((NPAD, C), jnp.float32),
    )(aggp, denp, x, r16, wa, ba, skip11)


_R16_NP = np.zeros((16, C), np.float32)
for _h in range(H):
    _R16_NP[_h, _h * D:(_h + 1) * D] = 1.0


def _block_diag(mats):
    return jax.scipy.linalg.block_diag(*[mats[h] for h in range(H)])


def kernel(x_author, x_paper, ei_writes, ei_rev, Wk, bk, Wq, bq, Wv, bv,
           Wa, ba, skip, Arel, Mrel, prel):
    f32 = jnp.float32
    scale = np.float32(1.0 / np.sqrt(D))
    xs = [jnp.pad(x_author, ((0, NPAD - N), (0, 0))),
          jnp.pad(x_paper, ((0, NPAD - N), (0, 0)))]

    bda = []
    bdm = []
    for r in range(2):
        colscale = jnp.repeat(prel[r] * scale, D)
        bda.append(_block_diag(Arel[r]) * colscale[None, :])
        bdm.append(_block_diag(Mrel[r]))

    qkv = [_proj(xs[t], Wq[t], bq[t][None], Wk[t], bk[t][None],
                 Wv[t], bv[t][None], bda[t], bdm[t]) for t in range(2)]

    r16 = jnp.asarray(_R16_NP)
    r16t = r16.T
    pad_e = E_PAD - E

    parts = []
    for (s_t, d_t, ei) in ((0, 1, ei_writes), (1, 0, ei_rev)):
        src = jnp.pad(ei[0], (0, pad_e))
        dst = jnp.pad(ei[1], (0, pad_e))
        q_d, k_s, v_s = qkv[d_t][0], qkv[s_t][1], qkv[s_t][2]
        qg, kg, vg = _gather_kernel(q_d, k_s, v_s, src, dst)
        wv, ex = _exwv(qg, kg, vg, r16t, r16)
        agg1 = jax.ops.segment_sum(wv, dst, num_segments=NPAD)
        den1 = jax.ops.segment_sum(ex, dst, num_segments=NPAD)
        zagg = jnp.zeros((1, NPAD, C), f32)
        zden = jnp.zeros((1, NPAD, 16), f32)
        parts.append((jnp.concatenate([agg1[None], zagg]),
                      jnp.concatenate([den1[None], zden])))

    outs = []
    for t in range(2):
        aggp, denp = parts[1 - t]
        o = _out_stage(aggp, denp, xs[t], r16, Wa[t], ba[t][None],
                       skip[t].reshape(1, 1))
        outs.append(o[:N])
    return (outs[0], outs[1])
